# Initial kernel scaffold; baseline (speedup 1.0000x reference)
import jax
import jax.numpy as jnp
from jax.experimental import pallas as pl


def kernel(x, edge_index, W1, b1, W2, b2, W3, b3, W4, b4):
    raise NotImplementedError("write your pallas kernel here")


# speedup vs baseline: 17.2469x; 17.2469x over previous
import functools

import jax
import jax.numpy as jnp
from jax import lax
from jax.experimental import pallas as pl
from jax.experimental.pallas import tpu as pltpu
from jax.experimental.pallas import tpu_sc as plsc

N = 50000
E = 800000
IN_C = 100
HID = 64
OUT_C = 18

G = 125
EROWS = E // G
CH = 40
N_PAD = 50176
ROWS_PER_TILE = N_PAD // 16
BN = 2000
GRID = N // BN

_mesh = plsc.VectorSubcoreMesh(core_axis_name="c", subcore_axis_name="s")
_f32 = jnp.float32


@functools.partial(
    pl.kernel,
    out_type=jax.ShapeDtypeStruct((2, N_PAD, 8), _f32),
    mesh=_mesh,
    compiler_params=pltpu.CompilerParams(use_tc_tiling_on_sc=False),
    scratch_types=[
        pltpu.VMEM((CH, G), jnp.int32),
        pltpu.VMEM((G, 8), _f32),
        pltpu.VMEM_SHARED((N_PAD, 8), _f32),
        pltpu.SemaphoreType.DMA,
    ],
)
def _sc_deg(dst_hbm, zeros_hbm, ones_hbm, out_hbm, idx_v, ones_v, table, sem):
    cid = lax.axis_index("c")
    sid = lax.axis_index("s")
    r0 = sid * ROWS_PER_TILE
    pltpu.sync_copy(zeros_hbm, table.at[pl.ds(r0, ROWS_PER_TILE)])
    pltpu.sync_copy(ones_hbm, ones_v)
    plsc.subcore_barrier()

    base = cid * (EROWS // 2) + sid * (EROWS // 32)

    @pl.loop(0, EROWS // 32 // CH)
    def _chunks(ch):
        pltpu.sync_copy(dst_hbm.at[pl.ds(base + ch * CH, CH)], idx_v)

        @pl.loop(0, CH)
        def _groups(g):
            pltpu.sync_copy(ones_v, table.at[idx_v.at[g]], add=True)

    plsc.subcore_barrier()
    pltpu.sync_copy(table.at[pl.ds(r0, ROWS_PER_TILE)],
                    out_hbm.at[cid, pl.ds(r0, ROWS_PER_TILE)])


@functools.partial(
    pl.kernel,
    out_type=jax.ShapeDtypeStruct((2, N_PAD, 32), _f32),
    mesh=_mesh,
    compiler_params=pltpu.CompilerParams(use_tc_tiling_on_sc=False),
    scratch_types=[
        pltpu.VMEM((CH, G), jnp.int32),
        pltpu.VMEM((CH, G), jnp.int32),
        pltpu.VMEM((G, 32), _f32),
        pltpu.VMEM_SHARED((N_PAD, 32), _f32),
        pltpu.SemaphoreType.DMA,
    ],
)
def _sc_agg64(hp_hbm, src_hbm, srcN_hbm, dst_hbm, zeros_hbm, out_hbm,
              src_v, dst_v, rows_v, acc, sem):
    cid = lax.axis_index("c")
    sid = lax.axis_index("s")
    r0 = sid * ROWS_PER_TILE
    pltpu.sync_copy(zeros_hbm, acc.at[pl.ds(r0, ROWS_PER_TILE)])
    plsc.subcore_barrier()

    base = sid * (EROWS // 16)

    @pl.loop(0, EROWS // 16 // CH)
    def _chunks(ch):
        @pl.when(cid == 0)
        def _():
            pltpu.sync_copy(src_hbm.at[pl.ds(base + ch * CH, CH)], src_v)

        @pl.when(cid == 1)
        def _():
            pltpu.sync_copy(srcN_hbm.at[pl.ds(base + ch * CH, CH)], src_v)

        pltpu.sync_copy(dst_hbm.at[pl.ds(base + ch * CH, CH)], dst_v)

        @pl.loop(0, CH)
        def _groups(g):
            pltpu.async_copy(hp_hbm.at[src_v.at[g]], rows_v, sem).wait()
            pltpu.sync_copy(rows_v, acc.at[dst_v.at[g]], add=True)

    plsc.subcore_barrier()
    pltpu.sync_copy(acc.at[pl.ds(r0, ROWS_PER_TILE)],
                    out_hbm.at[cid, pl.ds(r0, ROWS_PER_TILE)])


@functools.partial(
    pl.kernel,
    out_type=jax.ShapeDtypeStruct((2, N_PAD, 32), _f32),
    mesh=_mesh,
    compiler_params=pltpu.CompilerParams(use_tc_tiling_on_sc=False),
    scratch_types=[
        pltpu.VMEM((CH, G), jnp.int32),
        pltpu.VMEM((CH, G), jnp.int32),
        pltpu.VMEM((G, 32), _f32),
        pltpu.VMEM_SHARED((N_PAD, 32), _f32),
        pltpu.SemaphoreType.DMA,
    ],
)
def _sc_agg32(hp_hbm, src_hbm, dst_hbm, zeros_hbm, out_hbm,
              src_v, dst_v, rows_v, acc, sem):
    cid = lax.axis_index("c")
    sid = lax.axis_index("s")
    r0 = sid * ROWS_PER_TILE
    pltpu.sync_copy(zeros_hbm, acc.at[pl.ds(r0, ROWS_PER_TILE)])
    plsc.subcore_barrier()

    base = cid * (EROWS // 2) + sid * (EROWS // 32)

    @pl.loop(0, EROWS // 32 // CH)
    def _chunks(ch):
        pltpu.sync_copy(src_hbm.at[pl.ds(base + ch * CH, CH)], src_v)
        pltpu.sync_copy(dst_hbm.at[pl.ds(base + ch * CH, CH)], dst_v)

        @pl.loop(0, CH)
        def _groups(g):
            pltpu.async_copy(hp_hbm.at[src_v.at[g]], rows_v, sem).wait()
            pltpu.sync_copy(rows_v, acc.at[dst_v.at[g]], add=True)

    plsc.subcore_barrier()
    pltpu.sync_copy(acc.at[pl.ds(r0, ROWS_PER_TILE)],
                    out_hbm.at[cid, pl.ds(r0, ROWS_PER_TILE)])


def _tc1_body(x_ref, w_ref, degp_ref, h_ref, hp_ref, dinv_ref):
    deg = degp_ref[0, :, 0:1] + degp_ref[1, :, 0:1] + 1.0
    dinv = lax.rsqrt(deg)
    dinv_ref[...] = dinv
    h = jnp.dot(x_ref[...], w_ref[...], preferred_element_type=_f32)
    h_ref[...] = h
    hp = h * dinv
    hp_ref[0, :, :] = hp[:, :32]
    hp_ref[1, :, :] = hp[:, 32:]


def _tc_mid_body(acc_ref, hprev_ref, dinv_ref, b_ref, w_ref, h_ref, hp_ref):
    dinv = dinv_ref[...]
    accf = jnp.concatenate([acc_ref[0, :, :], acc_ref[1, :, :]], axis=1)
    g = jax.nn.relu(accf * dinv + hprev_ref[...] * (dinv * dinv) + b_ref[...])
    h = jnp.dot(g, w_ref[...], preferred_element_type=_f32)
    h_ref[...] = h
    hp = h * dinv
    hp_ref[0, :, :] = hp[:, :32]
    hp_ref[1, :, :] = hp[:, 32:]


def _tc4_body(acc_ref, hprev_ref, dinv_ref, b_ref, w_ref, h_ref, hp_ref):
    dinv = dinv_ref[...]
    accf = jnp.concatenate([acc_ref[0, :, :], acc_ref[1, :, :]], axis=1)
    g = jax.nn.relu(accf * dinv + hprev_ref[...] * (dinv * dinv) + b_ref[...])
    h = jnp.dot(g, w_ref[...], preferred_element_type=_f32)
    h_ref[...] = h
    hp_ref[...] = h * dinv


def _tc5_body(acc_ref, hprev_ref, dinv_ref, b_ref, out_ref):
    dinv = dinv_ref[...]
    accf = acc_ref[0, :, :] + acc_ref[1, :, :]
    out_ref[...] = accf * dinv + hprev_ref[...] * (dinv * dinv) + b_ref[...]


def _row_spec(c):
    return pl.BlockSpec((BN, c), lambda i: (i, 0))


def _split_spec(c):
    return pl.BlockSpec((2, BN, c), lambda i: (0, i, 0))


def _full_spec(r, c):
    return pl.BlockSpec((r, c), lambda i: (0, 0))


def kernel(x, edge_index, W1, b1, W2, b2, W3, b3, W4, b4):
    src = edge_index[0].reshape(EROWS, G)
    srcN = (edge_index[0] + N).reshape(EROWS, G)
    dst = edge_index[1].reshape(EROWS, G)
    zeros32 = jnp.zeros((ROWS_PER_TILE, 32), _f32)
    zeros8 = jnp.zeros((ROWS_PER_TILE, 8), _f32)
    ones8 = jnp.ones((G, 8), _f32)
    W4p = jnp.pad(W4, ((0, 0), (0, 32 - OUT_C)))
    b4p = jnp.pad(b4, (0, 32 - OUT_C))

    degp = _sc_deg(dst, zeros8, ones8)

    tc1 = pl.pallas_call(
        _tc1_body,
        grid=(GRID,),
        in_specs=[_row_spec(IN_C), _full_spec(IN_C, HID), _split_spec(8)],
        out_specs=[_row_spec(HID), _split_spec(32), _row_spec(1)],
        out_shape=[jax.ShapeDtypeStruct((N, HID), _f32),
                   jax.ShapeDtypeStruct((2, N, 32), _f32),
                   jax.ShapeDtypeStruct((N, 1), _f32)],
    )
    h1, hp1, dinv = tc1(x, W1, degp)

    tc_mid = pl.pallas_call(
        _tc_mid_body,
        grid=(GRID,),
        in_specs=[_split_spec(32), _row_spec(HID), _row_spec(1),
                  _full_spec(1, HID), _full_spec(HID, HID)],
        out_specs=[_row_spec(HID), _split_spec(32)],
        out_shape=[jax.ShapeDtypeStruct((N, HID), _f32),
                   jax.ShapeDtypeStruct((2, N, 32), _f32)],
    )

    acc1 = _sc_agg64(hp1.reshape(2 * N, 32), src, srcN, dst, zeros32)
    h2, hp2 = tc_mid(acc1, h1, dinv, b1[None, :], W2)

    acc2 = _sc_agg64(hp2.reshape(2 * N, 32), src, srcN, dst, zeros32)
    h3, hp3 = tc_mid(acc2, h2, dinv, b2[None, :], W3)

    acc3 = _sc_agg64(hp3.reshape(2 * N, 32), src, srcN, dst, zeros32)
    tc4 = pl.pallas_call(
        _tc4_body,
        grid=(GRID,),
        in_specs=[_split_spec(32), _row_spec(HID), _row_spec(1),
                  _full_spec(1, HID), _full_spec(HID, 32)],
        out_specs=[_row_spec(32), _row_spec(32)],
        out_shape=[jax.ShapeDtypeStruct((N, 32), _f32),
                   jax.ShapeDtypeStruct((N, 32), _f32)],
    )
    h4, hp4 = tc4(acc3, h3, dinv, b3[None, :], W4p)

    acc4 = _sc_agg32(hp4, src, dst, zeros32)
    tc5 = pl.pallas_call(
        _tc5_body,
        grid=(GRID,),
        in_specs=[_split_spec(32), _row_spec(32), _row_spec(1),
                  _full_spec(1, 32)],
        out_specs=_row_spec(32),
        out_shape=jax.ShapeDtypeStruct((N, 32), _f32),
    )
    out = tc5(acc4, h4, dinv, b4p[None, :])
    return out[:, :OUT_C]


# speedup vs baseline: 17.2501x; 1.0002x over previous
import functools

import jax
import jax.numpy as jnp
from jax import lax
from jax.experimental import pallas as pl
from jax.experimental.pallas import tpu as pltpu
from jax.experimental.pallas import tpu_sc as plsc

N = 50000
E = 800000
IN_C = 100
HID = 64
OUT_C = 18

G = 125
EROWS = E // G
CH = 40
N_PAD = 50176
ROWS_PER_TILE = N_PAD // 16
BN = 2000
GRID = N // BN

_mesh = plsc.VectorSubcoreMesh(core_axis_name="c", subcore_axis_name="s")
_f32 = jnp.float32


@functools.partial(
    pl.kernel,
    out_type=jax.ShapeDtypeStruct((2, N_PAD, 8), _f32),
    mesh=_mesh,
    compiler_params=pltpu.CompilerParams(use_tc_tiling_on_sc=False),
    scratch_types=[
        pltpu.VMEM((CH, G), jnp.int32),
        pltpu.VMEM((G, 8), _f32),
        pltpu.VMEM_SHARED((N_PAD, 8), _f32),
        pltpu.SemaphoreType.DMA,
    ],
)
def _sc_deg(dst_hbm, zeros_hbm, ones_hbm, out_hbm, idx_v, ones_v, table, sem):
    cid = lax.axis_index("c")
    sid = lax.axis_index("s")
    r0 = sid * ROWS_PER_TILE
    pltpu.sync_copy(zeros_hbm, table.at[pl.ds(r0, ROWS_PER_TILE)])
    pltpu.sync_copy(ones_hbm, ones_v)
    plsc.subcore_barrier()

    base = cid * (EROWS // 2) + sid * (EROWS // 32)

    @pl.loop(0, EROWS // 32 // CH)
    def _chunks(ch):
        pltpu.sync_copy(dst_hbm.at[pl.ds(base + ch * CH, CH)], idx_v)

        @pl.loop(0, CH)
        def _groups(g):
            pltpu.sync_copy(ones_v, table.at[idx_v.at[g]], add=True)

    plsc.subcore_barrier()
    pltpu.sync_copy(table.at[pl.ds(r0, ROWS_PER_TILE)],
                    out_hbm.at[cid, pl.ds(r0, ROWS_PER_TILE)])


NBUF = 4


def _make_agg(split_edges):
    nch = (EROWS // 32) // CH if split_edges else (EROWS // 16) // CH

    def body(*refs):
        if split_edges:
            (hp_hbm, src_hbm, dst_hbm, zeros_hbm, out_hbm,
             src_v, dst_v, bufs, acc, sem) = refs
            srcN_hbm = None
        else:
            (hp_hbm, src_hbm, srcN_hbm, dst_hbm, zeros_hbm, out_hbm,
             src_v, dst_v, bufs, acc, sem) = refs
        cid = lax.axis_index("c")
        sid = lax.axis_index("s")
        r0 = sid * ROWS_PER_TILE
        pltpu.sync_copy(zeros_hbm, acc.at[pl.ds(r0, ROWS_PER_TILE)])
        plsc.subcore_barrier()
        if split_edges:
            base = cid * (EROWS // 2) + sid * (EROWS // 32)
        else:
            base = sid * (EROWS // 16)

        @pl.loop(0, nch)
        def _chunk(ch):
            row0 = base + ch * CH
            if split_edges:
                pltpu.sync_copy(src_hbm.at[pl.ds(row0, CH)], src_v)
            else:
                @pl.when(cid == 0)
                def _():
                    pltpu.sync_copy(src_hbm.at[pl.ds(row0, CH)], src_v)

                @pl.when(cid == 1)
                def _():
                    pltpu.sync_copy(srcN_hbm.at[pl.ds(row0, CH)], src_v)

            pltpu.sync_copy(dst_hbm.at[pl.ds(row0, CH)], dst_v)

            @plsc.parallel_loop(0, CH, 1, unroll=NBUF)
            def _group(g):
                k = g & (NBUF - 1)
                pltpu.async_copy(hp_hbm.at[src_v.at[g]], bufs.at[k], sem).wait()
                pltpu.sync_copy(bufs.at[k], acc.at[dst_v.at[g]], add=True)

        plsc.subcore_barrier()
        pltpu.sync_copy(acc.at[pl.ds(r0, ROWS_PER_TILE)],
                        out_hbm.at[cid, pl.ds(r0, ROWS_PER_TILE)])

    return pl.kernel(
        body,
        out_type=jax.ShapeDtypeStruct((2, N_PAD, 32), _f32),
        mesh=_mesh,
        compiler_params=pltpu.CompilerParams(use_tc_tiling_on_sc=False),
        scratch_types=[
            pltpu.VMEM((CH, G), jnp.int32),
            pltpu.VMEM((CH, G), jnp.int32),
            pltpu.VMEM((NBUF, G, 32), _f32),
            pltpu.VMEM_SHARED((N_PAD, 32), _f32),
            pltpu.SemaphoreType.DMA,
        ],
    )


_sc_agg64 = _make_agg(split_edges=False)
_sc_agg32 = _make_agg(split_edges=True)


def _tc1_body(x_ref, w_ref, degp_ref, h_ref, hp_ref, dinv_ref):
    deg = degp_ref[0, :, 0:1] + degp_ref[1, :, 0:1] + 1.0
    dinv = lax.rsqrt(deg)
    dinv_ref[...] = dinv
    h = jnp.dot(x_ref[...], w_ref[...], preferred_element_type=_f32)
    h_ref[...] = h
    hp = h * dinv
    hp_ref[0, :, :] = hp[:, :32]
    hp_ref[1, :, :] = hp[:, 32:]


def _tc_mid_body(acc_ref, hprev_ref, dinv_ref, b_ref, w_ref, h_ref, hp_ref):
    dinv = dinv_ref[...]
    accf = jnp.concatenate([acc_ref[0, :, :], acc_ref[1, :, :]], axis=1)
    g = jax.nn.relu(accf * dinv + hprev_ref[...] * (dinv * dinv) + b_ref[...])
    h = jnp.dot(g, w_ref[...], preferred_element_type=_f32)
    h_ref[...] = h
    hp = h * dinv
    hp_ref[0, :, :] = hp[:, :32]
    hp_ref[1, :, :] = hp[:, 32:]


def _tc4_body(acc_ref, hprev_ref, dinv_ref, b_ref, w_ref, h_ref, hp_ref):
    dinv = dinv_ref[...]
    accf = jnp.concatenate([acc_ref[0, :, :], acc_ref[1, :, :]], axis=1)
    g = jax.nn.relu(accf * dinv + hprev_ref[...] * (dinv * dinv) + b_ref[...])
    h = jnp.dot(g, w_ref[...], preferred_element_type=_f32)
    h_ref[...] = h
    hp_ref[...] = h * dinv


def _tc5_body(acc_ref, hprev_ref, dinv_ref, b_ref, out_ref):
    dinv = dinv_ref[...]
    accf = acc_ref[0, :, :] + acc_ref[1, :, :]
    out_ref[...] = accf * dinv + hprev_ref[...] * (dinv * dinv) + b_ref[...]


def _row_spec(c):
    return pl.BlockSpec((BN, c), lambda i: (i, 0))


def _split_spec(c):
    return pl.BlockSpec((2, BN, c), lambda i: (0, i, 0))


def _full_spec(r, c):
    return pl.BlockSpec((r, c), lambda i: (0, 0))


def kernel(x, edge_index, W1, b1, W2, b2, W3, b3, W4, b4):
    src = edge_index[0].reshape(EROWS, G)
    srcN = (edge_index[0] + N).reshape(EROWS, G)
    dst = edge_index[1].reshape(EROWS, G)
    zeros32 = jnp.zeros((ROWS_PER_TILE, 32), _f32)
    zeros8 = jnp.zeros((ROWS_PER_TILE, 8), _f32)
    ones8 = jnp.ones((G, 8), _f32)
    W4p = jnp.pad(W4, ((0, 0), (0, 32 - OUT_C)))
    b4p = jnp.pad(b4, (0, 32 - OUT_C))

    degp = _sc_deg(dst, zeros8, ones8)

    tc1 = pl.pallas_call(
        _tc1_body,
        grid=(GRID,),
        in_specs=[_row_spec(IN_C), _full_spec(IN_C, HID), _split_spec(8)],
        out_specs=[_row_spec(HID), _split_spec(32), _row_spec(1)],
        out_shape=[jax.ShapeDtypeStruct((N, HID), _f32),
                   jax.ShapeDtypeStruct((2, N, 32), _f32),
                   jax.ShapeDtypeStruct((N, 1), _f32)],
    )
    h1, hp1, dinv = tc1(x, W1, degp)

    tc_mid = pl.pallas_call(
        _tc_mid_body,
        grid=(GRID,),
        in_specs=[_split_spec(32), _row_spec(HID), _row_spec(1),
                  _full_spec(1, HID), _full_spec(HID, HID)],
        out_specs=[_row_spec(HID), _split_spec(32)],
        out_shape=[jax.ShapeDtypeStruct((N, HID), _f32),
                   jax.ShapeDtypeStruct((2, N, 32), _f32)],
    )

    acc1 = _sc_agg64(hp1.reshape(2 * N, 32), src, srcN, dst, zeros32)
    h2, hp2 = tc_mid(acc1, h1, dinv, b1[None, :], W2)

    acc2 = _sc_agg64(hp2.reshape(2 * N, 32), src, srcN, dst, zeros32)
    h3, hp3 = tc_mid(acc2, h2, dinv, b2[None, :], W3)

    acc3 = _sc_agg64(hp3.reshape(2 * N, 32), src, srcN, dst, zeros32)
    tc4 = pl.pallas_call(
        _tc4_body,
        grid=(GRID,),
        in_specs=[_split_spec(32), _row_spec(HID), _row_spec(1),
                  _full_spec(1, HID), _full_spec(HID, 32)],
        out_specs=[_row_spec(32), _row_spec(32)],
        out_shape=[jax.ShapeDtypeStruct((N, 32), _f32),
                   jax.ShapeDtypeStruct((N, 32), _f32)],
    )
    h4, hp4 = tc4(acc3, h3, dinv, b3[None, :], W4p)

    acc4 = _sc_agg32(hp4, src, dst, zeros32)
    tc5 = pl.pallas_call(
        _tc5_body,
        grid=(GRID,),
        in_specs=[_split_spec(32), _row_spec(32), _row_spec(1),
                  _full_spec(1, 32)],
        out_specs=_row_spec(32),
        out_shape=jax.ShapeDtypeStruct((N, 32), _f32),
    )
    out = tc5(acc4, h4, dinv, b4p[None, :])
    return out[:, :OUT_C]


# speedup vs baseline: 25.0831x; 1.4541x over previous
import functools

import jax
import jax.numpy as jnp
from jax import lax
from jax.experimental import pallas as pl
from jax.experimental.pallas import tpu as pltpu
from jax.experimental.pallas import tpu_sc as plsc

N = 50000
E = 800000
IN_C = 100
HID = 64
OUT_C = 18

G = 500
EROWS = E // G
CH = 10
N_PAD = 50176
ROWS_PER_TILE = N_PAD // 16
BN = 2000
GRID = N // BN

_mesh = plsc.VectorSubcoreMesh(core_axis_name="c", subcore_axis_name="s")
_f32 = jnp.float32


@functools.partial(
    pl.kernel,
    out_type=jax.ShapeDtypeStruct((2, N_PAD, 8), _f32),
    mesh=_mesh,
    compiler_params=pltpu.CompilerParams(use_tc_tiling_on_sc=False),
    scratch_types=[
        pltpu.VMEM((CH, G), jnp.int32),
        pltpu.VMEM((G, 8), _f32),
        pltpu.VMEM_SHARED((N_PAD, 8), _f32),
        pltpu.SemaphoreType.DMA,
    ],
)
def _sc_deg(dst_hbm, zeros_hbm, ones_hbm, out_hbm, idx_v, ones_v, table, sem):
    cid = lax.axis_index("c")
    sid = lax.axis_index("s")
    r0 = sid * ROWS_PER_TILE
    pltpu.sync_copy(zeros_hbm, table.at[pl.ds(r0, ROWS_PER_TILE)])
    pltpu.sync_copy(ones_hbm, ones_v)
    plsc.subcore_barrier()

    base = cid * (EROWS // 2) + sid * (EROWS // 32)

    @pl.loop(0, EROWS // 32 // CH)
    def _chunks(ch):
        pltpu.sync_copy(dst_hbm.at[pl.ds(base + ch * CH, CH)], idx_v)

        @pl.loop(0, CH)
        def _groups(g):
            pltpu.sync_copy(ones_v, table.at[idx_v.at[g]], add=True)

    plsc.subcore_barrier()
    pltpu.sync_copy(table.at[pl.ds(r0, ROWS_PER_TILE)],
                    out_hbm.at[cid, pl.ds(r0, ROWS_PER_TILE)])


NBUF = 1


def _make_agg(split_edges):
    nch = (EROWS // 32) // CH if split_edges else (EROWS // 16) // CH

    def body(*refs):
        if split_edges:
            (hp_hbm, src_hbm, dst_hbm, zeros_hbm, out_hbm,
             src_v, dst_v, bufs, acc, sem) = refs
            srcN_hbm = None
        else:
            (hp_hbm, src_hbm, srcN_hbm, dst_hbm, zeros_hbm, out_hbm,
             src_v, dst_v, bufs, acc, sem) = refs
        cid = lax.axis_index("c")
        sid = lax.axis_index("s")
        r0 = sid * ROWS_PER_TILE
        pltpu.sync_copy(zeros_hbm, acc.at[pl.ds(r0, ROWS_PER_TILE)])
        plsc.subcore_barrier()
        if split_edges:
            base = cid * (EROWS // 2) + sid * (EROWS // 32)
        else:
            base = sid * (EROWS // 16)

        @pl.loop(0, nch)
        def _chunk(ch):
            row0 = base + ch * CH
            if split_edges:
                pltpu.sync_copy(src_hbm.at[pl.ds(row0, CH)], src_v)
            else:
                @pl.when(cid == 0)
                def _():
                    pltpu.sync_copy(src_hbm.at[pl.ds(row0, CH)], src_v)

                @pl.when(cid == 1)
                def _():
                    pltpu.sync_copy(srcN_hbm.at[pl.ds(row0, CH)], src_v)

            pltpu.sync_copy(dst_hbm.at[pl.ds(row0, CH)], dst_v)

            @pl.loop(0, CH)
            def _group(g):
                pltpu.async_copy(hp_hbm.at[src_v.at[g]], bufs.at[0], sem).wait()
                pltpu.sync_copy(bufs.at[0], acc.at[dst_v.at[g]], add=True)

        plsc.subcore_barrier()
        pltpu.sync_copy(acc.at[pl.ds(r0, ROWS_PER_TILE)],
                        out_hbm.at[cid, pl.ds(r0, ROWS_PER_TILE)])

    return pl.kernel(
        body,
        out_type=jax.ShapeDtypeStruct((2, N_PAD, 32), _f32),
        mesh=_mesh,
        compiler_params=pltpu.CompilerParams(use_tc_tiling_on_sc=False),
        scratch_types=[
            pltpu.VMEM((CH, G), jnp.int32),
            pltpu.VMEM((CH, G), jnp.int32),
            pltpu.VMEM((NBUF, G, 32), _f32),
            pltpu.VMEM_SHARED((N_PAD, 32), _f32),
            pltpu.SemaphoreType.DMA,
        ],
    )


_sc_agg64 = _make_agg(split_edges=False)
_sc_agg32 = _make_agg(split_edges=True)


def _tc1_body(x_ref, w_ref, degp_ref, h_ref, hp_ref, dinv_ref):
    deg = degp_ref[0, :, 0:1] + degp_ref[1, :, 0:1] + 1.0
    dinv = lax.rsqrt(deg)
    dinv_ref[...] = dinv
    h = jnp.dot(x_ref[...], w_ref[...], preferred_element_type=_f32)
    h_ref[...] = h
    hp = h * dinv
    hp_ref[0, :, :] = hp[:, :32]
    hp_ref[1, :, :] = hp[:, 32:]


def _tc_mid_body(acc_ref, hprev_ref, dinv_ref, b_ref, w_ref, h_ref, hp_ref):
    dinv = dinv_ref[...]
    accf = jnp.concatenate([acc_ref[0, :, :], acc_ref[1, :, :]], axis=1)
    g = jax.nn.relu(accf * dinv + hprev_ref[...] * (dinv * dinv) + b_ref[...])
    h = jnp.dot(g, w_ref[...], preferred_element_type=_f32)
    h_ref[...] = h
    hp = h * dinv
    hp_ref[0, :, :] = hp[:, :32]
    hp_ref[1, :, :] = hp[:, 32:]


def _tc4_body(acc_ref, hprev_ref, dinv_ref, b_ref, w_ref, h_ref, hp_ref):
    dinv = dinv_ref[...]
    accf = jnp.concatenate([acc_ref[0, :, :], acc_ref[1, :, :]], axis=1)
    g = jax.nn.relu(accf * dinv + hprev_ref[...] * (dinv * dinv) + b_ref[...])
    h = jnp.dot(g, w_ref[...], preferred_element_type=_f32)
    h_ref[...] = h
    hp_ref[...] = h * dinv


def _tc5_body(acc_ref, hprev_ref, dinv_ref, b_ref, out_ref):
    dinv = dinv_ref[...]
    accf = acc_ref[0, :, :] + acc_ref[1, :, :]
    out_ref[...] = accf * dinv + hprev_ref[...] * (dinv * dinv) + b_ref[...]


def _row_spec(c):
    return pl.BlockSpec((BN, c), lambda i: (i, 0))


def _split_spec(c):
    return pl.BlockSpec((2, BN, c), lambda i: (0, i, 0))


def _full_spec(r, c):
    return pl.BlockSpec((r, c), lambda i: (0, 0))


def kernel(x, edge_index, W1, b1, W2, b2, W3, b3, W4, b4):
    src = edge_index[0].reshape(EROWS, G)
    srcN = (edge_index[0] + N).reshape(EROWS, G)
    dst = edge_index[1].reshape(EROWS, G)
    zeros32 = jnp.zeros((ROWS_PER_TILE, 32), _f32)
    zeros8 = jnp.zeros((ROWS_PER_TILE, 8), _f32)
    ones8 = jnp.ones((G, 8), _f32)
    W4p = jnp.pad(W4, ((0, 0), (0, 32 - OUT_C)))
    b4p = jnp.pad(b4, (0, 32 - OUT_C))

    degp = _sc_deg(dst, zeros8, ones8)

    tc1 = pl.pallas_call(
        _tc1_body,
        grid=(GRID,),
        in_specs=[_row_spec(IN_C), _full_spec(IN_C, HID), _split_spec(8)],
        out_specs=[_row_spec(HID), _split_spec(32), _row_spec(1)],
        out_shape=[jax.ShapeDtypeStruct((N, HID), _f32),
                   jax.ShapeDtypeStruct((2, N, 32), _f32),
                   jax.ShapeDtypeStruct((N, 1), _f32)],
    )
    h1, hp1, dinv = tc1(x, W1, degp)

    tc_mid = pl.pallas_call(
        _tc_mid_body,
        grid=(GRID,),
        in_specs=[_split_spec(32), _row_spec(HID), _row_spec(1),
                  _full_spec(1, HID), _full_spec(HID, HID)],
        out_specs=[_row_spec(HID), _split_spec(32)],
        out_shape=[jax.ShapeDtypeStruct((N, HID), _f32),
                   jax.ShapeDtypeStruct((2, N, 32), _f32)],
    )

    acc1 = _sc_agg64(hp1.reshape(2 * N, 32), src, srcN, dst, zeros32)
    h2, hp2 = tc_mid(acc1, h1, dinv, b1[None, :], W2)

    acc2 = _sc_agg64(hp2.reshape(2 * N, 32), src, srcN, dst, zeros32)
    h3, hp3 = tc_mid(acc2, h2, dinv, b2[None, :], W3)

    acc3 = _sc_agg64(hp3.reshape(2 * N, 32), src, srcN, dst, zeros32)
    tc4 = pl.pallas_call(
        _tc4_body,
        grid=(GRID,),
        in_specs=[_split_spec(32), _row_spec(HID), _row_spec(1),
                  _full_spec(1, HID), _full_spec(HID, 32)],
        out_specs=[_row_spec(32), _row_spec(32)],
        out_shape=[jax.ShapeDtypeStruct((N, 32), _f32),
                   jax.ShapeDtypeStruct((N, 32), _f32)],
    )
    h4, hp4 = tc4(acc3, h3, dinv, b3[None, :], W4p)

    acc4 = _sc_agg32(hp4, src, dst, zeros32)
    tc5 = pl.pallas_call(
        _tc5_body,
        grid=(GRID,),
        in_specs=[_split_spec(32), _row_spec(32), _row_spec(1),
                  _full_spec(1, 32)],
        out_specs=_row_spec(32),
        out_shape=jax.ShapeDtypeStruct((N, 32), _f32),
    )
    out = tc5(acc4, h4, dinv, b4p[None, :])
    return out[:, :OUT_C]
